# baseline (speedup 1.0000x reference)
import functools

import jax
import jax.numpy as jnp
from jax import lax
from jax.experimental import pallas as pl
from jax.experimental.pallas import tpu as pltpu
from jax.experimental.pallas import tpu_sc as plsc

B = 16384
T = 50
D = 64
VOCAB = 100000
OUT = 128
LANES = 16
NC = 2
NS = 16
NW = NC * NS
NSPLIT = 1
BH = B // NSPLIT
ROWS_PER_W = BH // NW
RB = 16
STEPS = ROWS_PER_W // RB
IDX_CH = 80
NG = (RB * T) // IDX_CH
NJ = D // LANES

_PERM = (list(range(16)) + list(range(32, 48))
         + list(range(16, 32)) + list(range(48, 64)))


def _make_sc_pool():
    mesh = plsc.VectorSubcoreMesh(core_axis_name="c", subcore_axis_name="s")

    @functools.partial(
        pl.kernel,
        out_type=jax.ShapeDtypeStruct((BH, D), jnp.float32),
        mesh=mesh,
        compiler_params=pltpu.CompilerParams(
            use_tc_tiling_on_sc=False, needs_layout_passes=False),
        scratch_types=[
            pltpu.VMEM((ROWS_PER_W * T,), jnp.int32),
            pltpu.VMEM((RB * T, D // 2), jnp.float32),
            pltpu.VMEM((RB * T, D // 2), jnp.float32),
            pltpu.VMEM((RB, D), jnp.float32),
            pltpu.VMEM((RB, D), jnp.float32),
            pltpu.SemaphoreType.DMA,
            pltpu.SemaphoreType.DMA,
            pltpu.SemaphoreType.DMA,
            pltpu.SemaphoreType.DMA,
        ],
    )
    def sc_pool(ids_hbm, table_hbm, out_hbm, idx_all, rows0, rows1,
                outv0, outv1, semg0, semg1, semo0, semo1):
        wid = lax.axis_index("s") * NC + lax.axis_index("c")
        idx_base = wid * ROWS_PER_W * T
        out_base = wid * ROWS_PER_W

        pltpu.sync_copy(
            ids_hbm.at[pl.ds(idx_base, ROWS_PER_W * T)], idx_all)

        def fire(rows_ref, sem, s):
            for g in range(NG):
                pltpu.make_async_copy(
                    table_hbm.at[idx_all.at[
                        pl.ds(s * (RB * T) + g * IDX_CH, IDX_CH)]],
                    rows_ref.at[pl.ds(g * IDX_CH, IDX_CH)],
                    sem,
                ).start()

        def drain(rows_ref, sem):
            for g in range(NG):
                pltpu.make_async_copy(
                    table_hbm.at[idx_all.at[pl.ds(g * IDX_CH, IDX_CH)]],
                    rows_ref.at[pl.ds(g * IDX_CH, IDX_CH)],
                    sem,
                ).wait()

        def out_copy(out_v, semo, s):
            return pltpu.make_async_copy(
                out_v, out_hbm.at[pl.ds(out_base + s * RB, RB)], semo)

        def process(rows_ref, out_v, semo, s, wait_from):
            drain(rows_ref, [semg0, semg1][wait_from])

            @pl.when(s >= 2)
            def _():
                out_copy(out_v, semo, s).wait()

            @pl.loop(0, RB)
            def _(r):
                row0 = r * T

                def add_row(t, acc):
                    lo = plsc.bitcast(rows_ref[t, pl.ds(0, LANES)],
                                      jnp.bfloat16)
                    hi = plsc.bitcast(rows_ref[t, pl.ds(LANES, LANES)],
                                      jnp.bfloat16)
                    e0, o0 = plsc.unpack(lo, format=plsc.PackFormat.INTERLEAVED)
                    e1, o1 = plsc.unpack(hi, format=plsc.PackFormat.INTERLEAVED)
                    return (acc[0] + e0, acc[1] + o0, acc[2] + e1, acc[3] + o1)

                def body(i, acc):
                    t4 = row0 + 4 * i
                    return add_row(
                        t4 + 3, add_row(t4 + 2, add_row(t4 + 1,
                                                        add_row(t4, acc))))

                zero = jnp.zeros((LANES,), jnp.float32)
                init = add_row(row0 + T - 1,
                               add_row(row0 + T - 2,
                                       (zero, zero, zero, zero)))
                acc = lax.fori_loop(0, (T - 2) // 4, body, init)
                for j in range(NJ):
                    out_v[r, pl.ds(j * LANES, LANES)] = acc[j]

            out_copy(out_v, semo, s).start()

        fire(rows0, semg0, 0)
        fire(rows1, semg1, 1)

        @pl.loop(0, STEPS, step=2)
        def _(s):
            process(rows0, outv0, semo0, s, 0)

            @pl.when(s + 2 < STEPS)
            def _():
                fire(rows0, semg0, s + 2)
            process(rows1, outv1, semo1, s + 1, 1)

            @pl.when(s + 3 < STEPS)
            def _():
                fire(rows1, semg1, s + 3)

        out_copy(outv0, semo0, 0).wait()
        out_copy(outv1, semo1, 0).wait()

    return sc_pool


_sc_pool_cache = []


def _sc_pool(ids_flat, table_bf):
    if not _sc_pool_cache:
        _sc_pool_cache.append(_make_sc_pool())
    return _sc_pool_cache[0](ids_flat, table_bf)


PACK_BLK = 1024
TBLK = 4 * PACK_BLK
VPAD = 25 * TBLK


VTAIL = VOCAB - (VPAD // TBLK - 1) * TBLK


def _pack_w(x):
    u16 = jax.lax.bitcast_convert_type(x.astype(jnp.bfloat16), jnp.uint16)
    u32 = u16.astype(jnp.uint32)
    w = u32[:, :D // 2] | (u32[:, D // 2:] << 16)
    return jax.lax.bitcast_convert_type(w, jnp.float32)


def _pack_body(t_hbm, out_ref, buf, sem):
    i = pl.program_id(0)
    last = VPAD // TBLK - 1

    @pl.when(i < last)
    def _():
        pltpu.make_async_copy(
            t_hbm.at[pl.ds(i * TBLK, TBLK), :], buf, sem).start()
        pltpu.make_async_copy(
            t_hbm.at[pl.ds(i * TBLK, TBLK), :], buf, sem).wait()

    @pl.when(i == last)
    def _():
        cp = pltpu.make_async_copy(
            t_hbm.at[pl.ds(last * TBLK, VTAIL), :],
            buf.at[pl.ds(0, VTAIL), :], sem)
        cp.start()
        cp.wait()

    x = buf[...]
    for a in range(4):
        out_ref[:, a * (D // 2):(a + 1) * (D // 2)] = (
            _pack_w(x[a * PACK_BLK:(a + 1) * PACK_BLK]))


def _pack_table(table):
    return pl.pallas_call(
        _pack_body,
        grid=(VPAD // TBLK,),
        in_specs=[pl.BlockSpec(memory_space=pl.ANY)],
        out_specs=pl.BlockSpec((PACK_BLK, 4 * (D // 2)), lambda i: (i, 0)),
        out_shape=jax.ShapeDtypeStruct((VPAD // 4, 4 * (D // 2)),
                                       jnp.float32),
        scratch_shapes=[pltpu.VMEM((TBLK, D), jnp.float32),
                        pltpu.SemaphoreType.DMA],
    )(table)


RBLK = 2048


def _mlp_body(ids_ref, ps_ref, w1_ref, b1_ref, w2_ref, b2_ref, out_ref):
    ids = ids_ref[...]
    cnt = jnp.sum((ids != 0).astype(jnp.float32), axis=1, keepdims=True)
    denom = jnp.maximum(cnt, 1.0)
    pooled = ps_ref[...] / denom
    h = jnp.maximum(
        jnp.dot(pooled, w1_ref[...], preferred_element_type=jnp.float32)
        + b1_ref[...], 0.0)
    out_ref[...] = (
        jnp.dot(h, w2_ref[...], preferred_element_type=jnp.float32)
        + b2_ref[...])


def _tc_mlp(tag_ids, pooled_sum, W1, b1, W2, b2):
    nrows = tag_ids.shape[0]
    return pl.pallas_call(
        _mlp_body,
        grid=(nrows // RBLK,),
        in_specs=[
            pl.BlockSpec((RBLK, T), lambda i: (i, 0)),
            pl.BlockSpec((RBLK, D), lambda i: (i, 0)),
            pl.BlockSpec((D, D), lambda i: (0, 0)),
            pl.BlockSpec((1, D), lambda i: (0, 0)),
            pl.BlockSpec((D, OUT), lambda i: (0, 0)),
            pl.BlockSpec((1, OUT), lambda i: (0, 0)),
        ],
        out_specs=pl.BlockSpec((RBLK, OUT), lambda i: (i, 0)),
        out_shape=jax.ShapeDtypeStruct((nrows, OUT), jnp.float32),
    )(tag_ids, pooled_sum, W1, b1, W2, b2)


def kernel(tag_ids, table, W1, b1, W2, b2):
    ids_flat = tag_ids.reshape(B * T)
    ids_q = (((((ids_flat >> 12) << 10) | (ids_flat & 1023)) << 2)
             | ((ids_flat >> 10) & 3))
    table_sc = _pack_table(table).reshape(VPAD, D // 2)
    W1p = W1[jnp.array(_PERM), :]
    b1r = b1.reshape(1, D)
    b2r = b2.reshape(1, OUT)
    pooled = _sc_pool(ids_q, table_sc)
    return _tc_mlp(tag_ids, pooled, W1p, b1r, W2, b2r)


# speedup vs baseline: 1.1914x; 1.1914x over previous
import functools

import jax
import jax.numpy as jnp
from jax import lax
from jax.experimental import pallas as pl
from jax.experimental.pallas import tpu as pltpu
from jax.experimental.pallas import tpu_sc as plsc

B = 16384
T = 50
D = 64
VOCAB = 100000
OUT = 128
LANES = 16
NC = 2
NS = 16
NW = NC * NS
NSPLIT = 1
BH = B // NSPLIT
ROWS_PER_W = BH // NW
RB = 16
STEPS = ROWS_PER_W // RB
IDX_CH = 80
NG = (RB * T) // IDX_CH
NJ = D // LANES

_PERM = (list(range(16)) + list(range(32, 48))
         + list(range(16, 32)) + list(range(48, 64)))


def _make_sc_pool():
    mesh = plsc.VectorSubcoreMesh(core_axis_name="c", subcore_axis_name="s")

    @functools.partial(
        pl.kernel,
        out_type=jax.ShapeDtypeStruct((BH, D), jnp.float32),
        mesh=mesh,
        compiler_params=pltpu.CompilerParams(
            use_tc_tiling_on_sc=False, needs_layout_passes=False),
        scratch_types=[
            pltpu.VMEM((ROWS_PER_W * T,), jnp.int32),
            pltpu.VMEM((RB * T, D // 2), jnp.float32),
            pltpu.VMEM((RB * T, D // 2), jnp.float32),
            pltpu.VMEM((RB, D), jnp.float32),
            pltpu.VMEM((RB, D), jnp.float32),
            pltpu.SemaphoreType.DMA,
            pltpu.SemaphoreType.DMA,
            pltpu.SemaphoreType.DMA,
            pltpu.SemaphoreType.DMA,
        ],
    )
    def sc_pool(ids_hbm, table_hbm, out_hbm, idx_all, rows0, rows1,
                outv0, outv1, semg0, semg1, semo0, semo1):
        wid = lax.axis_index("s") * NC + lax.axis_index("c")
        idx_base = wid * ROWS_PER_W * T
        out_base = wid * ROWS_PER_W

        pltpu.sync_copy(
            ids_hbm.at[pl.ds(idx_base, ROWS_PER_W * T)], idx_all)

        def fire(rows_ref, sem, s):
            for g in range(NG):
                pltpu.make_async_copy(
                    table_hbm.at[idx_all.at[
                        pl.ds(s * (RB * T) + g * IDX_CH, IDX_CH)]],
                    rows_ref.at[pl.ds(g * IDX_CH, IDX_CH)],
                    sem,
                ).start()

        def drain(rows_ref, sem):
            for g in range(NG):
                pltpu.make_async_copy(
                    table_hbm.at[idx_all.at[pl.ds(g * IDX_CH, IDX_CH)]],
                    rows_ref.at[pl.ds(g * IDX_CH, IDX_CH)],
                    sem,
                ).wait()

        def out_copy(out_v, semo, s):
            return pltpu.make_async_copy(
                out_v, out_hbm.at[pl.ds(out_base + s * RB, RB)], semo)

        def process(rows_ref, out_v, semo, s, wait_from):
            drain(rows_ref, [semg0, semg1][wait_from])

            @pl.when(s >= 2)
            def _():
                out_copy(out_v, semo, s).wait()

            @pl.loop(0, RB)
            def _(r):
                row0 = r * T

                def add_row(t, acc):
                    lo = plsc.bitcast(rows_ref[t, pl.ds(0, LANES)],
                                      jnp.bfloat16)
                    hi = plsc.bitcast(rows_ref[t, pl.ds(LANES, LANES)],
                                      jnp.bfloat16)
                    e0, o0 = plsc.unpack(lo, format=plsc.PackFormat.INTERLEAVED)
                    e1, o1 = plsc.unpack(hi, format=plsc.PackFormat.INTERLEAVED)
                    return (acc[0] + e0, acc[1] + o0, acc[2] + e1, acc[3] + o1)

                def body(i, acc):
                    t4 = row0 + 4 * i
                    return add_row(
                        t4 + 3, add_row(t4 + 2, add_row(t4 + 1,
                                                        add_row(t4, acc))))

                zero = jnp.zeros((LANES,), jnp.float32)
                init = add_row(row0 + T - 1,
                               add_row(row0 + T - 2,
                                       (zero, zero, zero, zero)))
                acc = lax.fori_loop(0, (T - 2) // 4, body, init)
                for j in range(NJ):
                    out_v[r, pl.ds(j * LANES, LANES)] = acc[j]

            out_copy(out_v, semo, s).start()

        fire(rows0, semg0, 0)
        fire(rows1, semg1, 1)

        @pl.loop(0, STEPS, step=2)
        def _(s):
            process(rows0, outv0, semo0, s, 0)

            @pl.when(s + 2 < STEPS)
            def _():
                fire(rows0, semg0, s + 2)
            process(rows1, outv1, semo1, s + 1, 1)

            @pl.when(s + 3 < STEPS)
            def _():
                fire(rows1, semg1, s + 3)

        out_copy(outv0, semo0, 0).wait()
        out_copy(outv1, semo1, 0).wait()

    return sc_pool


_sc_pool_cache = []


def _sc_pool(ids_flat, table_bf):
    if not _sc_pool_cache:
        _sc_pool_cache.append(_make_sc_pool())
    return _sc_pool_cache[0](ids_flat, table_bf)


PACK_BLK = 1024
TBLK = 4 * PACK_BLK
VPAD = 25 * TBLK


VTAIL = VOCAB - (VPAD // TBLK - 1) * TBLK


def _pack_w(x):
    u16 = jax.lax.bitcast_convert_type(x.astype(jnp.bfloat16), jnp.uint16)
    u32 = u16.astype(jnp.uint32)
    w = u32[:, :D // 2] | (u32[:, D // 2:] << 16)
    return jax.lax.bitcast_convert_type(w, jnp.float32)


def _fetch(t_hbm, buf, sem, i):
    last = VPAD // TBLK - 1

    @pl.when(i < last)
    def _():
        pltpu.make_async_copy(
            t_hbm.at[pl.ds(i * TBLK, TBLK), :], buf, sem).start()

    @pl.when(i == last)
    def _():
        pltpu.make_async_copy(
            t_hbm.at[pl.ds(last * TBLK, VTAIL), :],
            buf.at[pl.ds(0, VTAIL), :], sem).start()


def _fetch_wait(t_hbm, buf, sem, i):
    last = VPAD // TBLK - 1

    @pl.when(i < last)
    def _():
        pltpu.make_async_copy(
            t_hbm.at[pl.ds(0, TBLK), :], buf, sem).wait()

    @pl.when(i == last)
    def _():
        pltpu.make_async_copy(
            t_hbm.at[pl.ds(0, VTAIL), :],
            buf.at[pl.ds(0, VTAIL), :], sem).wait()


def _pack_body(t_hbm, out_ref, buf0, buf1, sem0, sem1):
    i = pl.program_id(0)

    @pl.when(i == 0)
    def _():
        _fetch(t_hbm, buf0, sem0, 0)
        _fetch(t_hbm, buf1, sem1, 1)

    def emit(buf, sem):
        _fetch_wait(t_hbm, buf, sem, i)
        x = buf[...]
        for a in range(4):
            out_ref[:, a * (D // 2):(a + 1) * (D // 2)] = (
                _pack_w(x[a * PACK_BLK:(a + 1) * PACK_BLK]))

        @pl.when(i + 2 < VPAD // TBLK)
        def _():
            _fetch(t_hbm, buf, sem, i + 2)

    @pl.when(i % 2 == 0)
    def _():
        emit(buf0, sem0)

    @pl.when(i % 2 == 1)
    def _():
        emit(buf1, sem1)


def _pack_table(table):
    return pl.pallas_call(
        _pack_body,
        grid=(VPAD // TBLK,),
        in_specs=[pl.BlockSpec(memory_space=pl.ANY)],
        out_specs=pl.BlockSpec((PACK_BLK, 4 * (D // 2)), lambda i: (i, 0)),
        out_shape=jax.ShapeDtypeStruct((VPAD // 4, 4 * (D // 2)),
                                       jnp.float32),
        scratch_shapes=[pltpu.VMEM((TBLK, D), jnp.float32),
                        pltpu.VMEM((TBLK, D), jnp.float32),
                        pltpu.SemaphoreType.DMA,
                        pltpu.SemaphoreType.DMA],
    )(table)


RBLK = 2048


def _mlp_body(ids_ref, ps_ref, w1_ref, b1_ref, w2_ref, b2_ref, out_ref):
    ids = ids_ref[...]
    cnt = jnp.sum((ids != 0).astype(jnp.float32), axis=1, keepdims=True)
    denom = jnp.maximum(cnt, 1.0)
    pooled = ps_ref[...] / denom
    h = jnp.maximum(
        jnp.dot(pooled, w1_ref[...], preferred_element_type=jnp.float32)
        + b1_ref[...], 0.0)
    out_ref[...] = (
        jnp.dot(h, w2_ref[...], preferred_element_type=jnp.float32)
        + b2_ref[...])


def _tc_mlp(tag_ids, pooled_sum, W1, b1, W2, b2):
    nrows = tag_ids.shape[0]
    return pl.pallas_call(
        _mlp_body,
        grid=(nrows // RBLK,),
        in_specs=[
            pl.BlockSpec((RBLK, T), lambda i: (i, 0)),
            pl.BlockSpec((RBLK, D), lambda i: (i, 0)),
            pl.BlockSpec((D, D), lambda i: (0, 0)),
            pl.BlockSpec((1, D), lambda i: (0, 0)),
            pl.BlockSpec((D, OUT), lambda i: (0, 0)),
            pl.BlockSpec((1, OUT), lambda i: (0, 0)),
        ],
        out_specs=pl.BlockSpec((RBLK, OUT), lambda i: (i, 0)),
        out_shape=jax.ShapeDtypeStruct((nrows, OUT), jnp.float32),
    )(tag_ids, pooled_sum, W1, b1, W2, b2)


def kernel(tag_ids, table, W1, b1, W2, b2):
    ids_flat = tag_ids.reshape(B * T)
    ids_q = (((((ids_flat >> 12) << 10) | (ids_flat & 1023)) << 2)
             | ((ids_flat >> 10) & 3))
    table_sc = _pack_table(table).reshape(VPAD, D // 2)
    W1p = W1[jnp.array(_PERM), :]
    b1r = b1.reshape(1, D)
    b2r = b2.reshape(1, OUT)
    pooled = _sc_pool(ids_q, table_sc)
    return _tc_mlp(tag_ids, pooled, W1p, b1r, W2, b2r)
